# Initial kernel scaffold; baseline (speedup 1.0000x reference)
import jax
import jax.numpy as jnp
from jax.experimental import pallas as pl


def kernel(x_num, x_cat, weight, bias, emb_table, category_offsets):
    raise NotImplementedError("write your pallas kernel here")


# speedup vs baseline: 1.2556x; 1.2556x over previous
import functools

import jax
import jax.numpy as jnp
from jax import lax
from jax.experimental import pallas as pl
from jax.experimental.pallas import tpu as pltpu
from jax.experimental.pallas import tpu_sc as plsc

B = 16384
D_NUM = 13
N_CAT = 26
D_TOKEN = 32
N_TOK = 1 + D_NUM + N_CAT

NW = 32
RPT = B // NW
CB = 32
NCH = RPT // CB

_mesh = plsc.VectorSubcoreMesh(core_axis_name="c", subcore_axis_name="s")


@functools.partial(
    pl.kernel,
    mesh=_mesh,
    compiler_params=pltpu.CompilerParams(use_tc_tiling_on_sc=False),
    out_type=jax.ShapeDtypeStruct((B, N_TOK, D_TOKEN), jnp.float32),
    scratch_types=[
        pltpu.VMEM((CB, N_CAT), jnp.int32),
        pltpu.VMEM((CB, 16), jnp.float32),
        pltpu.VMEM((1 + D_NUM, D_TOKEN), jnp.float32),
        pltpu.VMEM((D_NUM + N_CAT, D_TOKEN), jnp.float32),
        pltpu.VMEM((CB, N_TOK, D_TOKEN), jnp.float32),
        pltpu.SemaphoreType.DMA,
    ],
)
def _tokenizer_sc(table, idxh, xnh, wh, bh, outh,
                  idx_v, xn_v, w_v, b_v, out_v, sem):
    wid = lax.axis_index("s") * 2 + lax.axis_index("c")
    pltpu.sync_copy(wh, w_v)
    pltpu.sync_copy(bh, b_v)

    def chunk_body(ch, carry):
        b0 = wid * RPT + ch * CB

        pltpu.sync_copy(idxh.at[pl.ds(b0, CB)], idx_v)
        pltpu.sync_copy(xnh.at[pl.ds(b0, CB)], xn_v)

        def fire(b, c):
            pltpu.async_copy(table.at[idx_v.at[b]],
                             out_v.at[b, pl.ds(14, N_CAT)], sem)
            return c
        lax.fori_loop(0, CB, fire, 0)
        def drain(b, c):
            pltpu.make_async_copy(table.at[idx_v.at[b]],
                                  out_v.at[b, pl.ds(14, N_CAT)], sem).wait()
            return c
        lax.fori_loop(0, CB, drain, 0)

        for c in range(N_CAT):
            bl = b_v[D_NUM + c, pl.ds(0, 16)]
            bh2 = b_v[D_NUM + c, pl.ds(16, 16)]

            def addb(b, cc, bl=bl, bh2=bh2, c=c):
                r = 14 + c
                out_v[b, r, pl.ds(0, 16)] = out_v[b, r, pl.ds(0, 16)] + bl
                out_v[b, r, pl.ds(16, 16)] = out_v[b, r, pl.ds(16, 16)] + bh2
                return cc
            lax.fori_loop(0, CB, addb, 0)

        w0l = w_v[0, pl.ds(0, 16)]
        w0h = w_v[0, pl.ds(16, 16)]

        def row0(b, cc):
            out_v[b, 0, pl.ds(0, 16)] = w0l
            out_v[b, 0, pl.ds(16, 16)] = w0h
            return cc
        lax.fori_loop(0, CB, row0, 0)

        for j in range(D_NUM):
            wl = w_v[1 + j, pl.ds(0, 16)]
            wh2 = w_v[1 + j, pl.ds(16, 16)]
            bl = b_v[j, pl.ds(0, 16)]
            bh2 = b_v[j, pl.ds(16, 16)]

            def numb(b, cc, wl=wl, wh2=wh2, bl=bl, bh2=bh2, j=j):
                s = xn_v[b, pl.ds(0, 16)][j]
                out_v[b, 1 + j, pl.ds(0, 16)] = wl * s + bl
                out_v[b, 1 + j, pl.ds(16, 16)] = wh2 * s + bh2
                return cc
            lax.fori_loop(0, CB, numb, 0)

        pltpu.sync_copy(out_v, outh.at[pl.ds(b0, CB)])
        return carry

    lax.fori_loop(0, NCH, chunk_body, 0)


def kernel(x_num, x_cat, weight, bias, emb_table, category_offsets):
    idx = x_cat.astype(jnp.int32) + category_offsets[None, :].astype(jnp.int32)
    xn = jnp.pad(x_num, ((0, 0), (0, 16 - D_NUM)))
    return _tokenizer_sc(emb_table, idx, xn, weight, bias)


# speedup vs baseline: 1.3569x; 1.0807x over previous
import functools

import jax
import jax.numpy as jnp
from jax import lax
from jax.experimental import pallas as pl
from jax.experimental.pallas import tpu as pltpu
from jax.experimental.pallas import tpu_sc as plsc

B = 16384
D_NUM = 13
N_CAT = 26
D_TOKEN = 32
N_TOK = 1 + D_NUM + N_CAT

NW = 32
RPT = B // NW
CB = 32
NCH = RPT // CB
G = 104
NG = CB * N_CAT // G

_mesh = plsc.VectorSubcoreMesh(core_axis_name="c", subcore_axis_name="s")


@functools.partial(
    pl.kernel,
    mesh=_mesh,
    compiler_params=pltpu.CompilerParams(use_tc_tiling_on_sc=False),
    out_type=jax.ShapeDtypeStruct((B, N_TOK, D_TOKEN), jnp.float32),
    scratch_types=[
        pltpu.VMEM((CB * N_CAT,), jnp.int32),
        pltpu.VMEM((CB, 16), jnp.float32),
        pltpu.VMEM((1 + D_NUM, D_TOKEN), jnp.float32),
        pltpu.VMEM((D_NUM + N_CAT, D_TOKEN), jnp.float32),
        pltpu.VMEM((CB * N_CAT, D_TOKEN), jnp.float32),
        pltpu.VMEM((CB, N_TOK, D_TOKEN), jnp.float32),
        pltpu.SemaphoreType.DMA,
    ],
)
def _tokenizer_sc(table, idxh, xnh, wh, bh, outh,
                  idx_v, xn_v, w_v, b_v, cat_v, out_v, sem):
    wid = lax.axis_index("s") * 2 + lax.axis_index("c")
    pltpu.sync_copy(wh, w_v)
    pltpu.sync_copy(bh, b_v)

    def chunk_body(ch, carry):
        b0 = wid * RPT + ch * CB
        i0 = b0 * N_CAT

        pltpu.sync_copy(idxh.at[pl.ds(i0, CB * N_CAT)], idx_v)
        pltpu.sync_copy(xnh.at[pl.ds(b0, CB)], xn_v)

        for k in range(NG):
            pltpu.async_copy(table.at[idx_v.at[pl.ds(k * G, G)]],
                             cat_v.at[pl.ds(k * G, G)], sem)

        w0l = w_v[0, pl.ds(0, 16)]
        w0h = w_v[0, pl.ds(16, 16)]

        @plsc.parallel_loop(0, CB)
        def _numeric(b):
            out_v[b, 0, pl.ds(0, 16)] = w0l
            out_v[b, 0, pl.ds(16, 16)] = w0h
            xr = xn_v[b, pl.ds(0, 16)]
            for j in range(D_NUM):
                s = xr[j]
                out_v[b, 1 + j, pl.ds(0, 16)] = (
                    w_v[1 + j, pl.ds(0, 16)] * s + b_v[j, pl.ds(0, 16)])
                out_v[b, 1 + j, pl.ds(16, 16)] = (
                    w_v[1 + j, pl.ds(16, 16)] * s + b_v[j, pl.ds(16, 16)])

        for k in range(NG):
            pltpu.make_async_copy(table.at[idx_v.at[pl.ds(k * G, G)]],
                                  cat_v.at[pl.ds(k * G, G)], sem).wait()

        @plsc.parallel_loop(0, CB)
        def _biasadd(b):
            r0 = b * N_CAT
            for c in range(N_CAT):
                out_v[b, 14 + c, pl.ds(0, 16)] = (
                    cat_v[r0 + c, pl.ds(0, 16)] + b_v[D_NUM + c, pl.ds(0, 16)])
                out_v[b, 14 + c, pl.ds(16, 16)] = (
                    cat_v[r0 + c, pl.ds(16, 16)] + b_v[D_NUM + c, pl.ds(16, 16)])

        pltpu.sync_copy(out_v, outh.at[pl.ds(b0, CB)])
        return carry

    lax.fori_loop(0, NCH, chunk_body, 0)


def kernel(x_num, x_cat, weight, bias, emb_table, category_offsets):
    idx = x_cat.astype(jnp.int32) + category_offsets[None, :].astype(jnp.int32)
    xn = jnp.pad(x_num, ((0, 0), (0, 16 - D_NUM)))
    return _tokenizer_sc(emb_table, idx.reshape(-1), xn, weight, bias)


# speedup vs baseline: 1.3889x; 1.0235x over previous
import functools

import jax
import jax.numpy as jnp
from jax import lax
from jax.experimental import pallas as pl
from jax.experimental.pallas import tpu as pltpu
from jax.experimental.pallas import tpu_sc as plsc

B = 16384
D_NUM = 13
N_CAT = 26
D_TOKEN = 32
N_TOK = 1 + D_NUM + N_CAT
ROW = N_TOK * D_TOKEN

NW = 32
RPT = B // NW
CB = 32
NCH = RPT // CB
NP = NCH // 2
G = 104
NG = CB * N_CAT // G

_mesh = plsc.VectorSubcoreMesh(core_axis_name="c", subcore_axis_name="s")


@functools.partial(
    pl.kernel,
    mesh=_mesh,
    compiler_params=pltpu.CompilerParams(use_tc_tiling_on_sc=False),
    out_type=jax.ShapeDtypeStruct((B * ROW,), jnp.float32),
    scratch_types=[
        pltpu.VMEM((CB * N_CAT,), jnp.int32),
        pltpu.VMEM((CB * N_CAT,), jnp.int32),
        pltpu.VMEM((CB, 16), jnp.float32),
        pltpu.VMEM((1 + D_NUM, D_TOKEN), jnp.float32),
        pltpu.VMEM((D_NUM + N_CAT, D_TOKEN), jnp.float32),
        pltpu.VMEM((CB * N_CAT, D_TOKEN), jnp.float32),
        pltpu.VMEM((CB * N_CAT, D_TOKEN), jnp.float32),
        pltpu.VMEM((CB * ROW,), jnp.float32),
        pltpu.SemaphoreType.DMA,
        pltpu.SemaphoreType.DMA,
        pltpu.SemaphoreType.DMA,
    ],
)
def _tokenizer_sc(table, idxh, xnh, wh, bh, outh,
                  idx0, idx1, xn_v, w_v, b_v, cat0, cat1, out_v,
                  semg0, semg1, semo):
    wid = lax.axis_index("s") * 2 + lax.axis_index("c")
    base = wid * RPT
    pltpu.sync_copy(wh, w_v)
    pltpu.sync_copy(bh, b_v)

    def prefetch(ch, idx_v, cat_v, semg):
        i0 = (base + ch * CB) * N_CAT
        pltpu.sync_copy(idxh.at[pl.ds(i0, CB * N_CAT)], idx_v)
        for k in range(NG):
            pltpu.async_copy(table.at[idx_v.at[pl.ds(k * G, G)]],
                             cat_v.at[pl.ds(k * G, G)], semg)

    def compute(ch, idx_v, cat_v, semg):
        b0 = base + ch * CB
        pltpu.sync_copy(xnh.at[pl.ds(b0, CB)], xn_v)

        @pl.when(ch >= 1)
        def _wait_prev_write():
            bp = (b0 - CB) * ROW
            pltpu.make_async_copy(out_v, outh.at[pl.ds(bp, CB * ROW)],
                                  semo).wait()

        w0l = w_v[0, pl.ds(0, 16)]
        w0h = w_v[0, pl.ds(16, 16)]

        @plsc.parallel_loop(0, CB)
        def _numeric(b):
            o = b * ROW
            out_v[pl.ds(o, 16)] = w0l
            out_v[pl.ds(o + 16, 16)] = w0h
            xr = xn_v[b, pl.ds(0, 16)]
            for j in range(D_NUM):
                s = xr[j]
                r = o + (1 + j) * D_TOKEN
                out_v[pl.ds(r, 16)] = w_v[1 + j, pl.ds(0, 16)] * s + b_v[j, pl.ds(0, 16)]
                out_v[pl.ds(r + 16, 16)] = (
                    w_v[1 + j, pl.ds(16, 16)] * s + b_v[j, pl.ds(16, 16)])

        for k in range(NG):
            pltpu.make_async_copy(table.at[idx_v.at[pl.ds(k * G, G)]],
                                  cat_v.at[pl.ds(k * G, G)], semg).wait()

        @plsc.parallel_loop(0, CB)
        def _biasadd(b):
            r0 = b * N_CAT
            o = b * ROW + (1 + D_NUM) * D_TOKEN
            for c in range(N_CAT):
                r = o + c * D_TOKEN
                out_v[pl.ds(r, 16)] = (
                    cat_v[r0 + c, pl.ds(0, 16)] + b_v[D_NUM + c, pl.ds(0, 16)])
                out_v[pl.ds(r + 16, 16)] = (
                    cat_v[r0 + c, pl.ds(16, 16)] + b_v[D_NUM + c, pl.ds(16, 16)])

        pltpu.async_copy(out_v, outh.at[pl.ds(b0 * ROW, CB * ROW)], semo)

    prefetch(0, idx0, cat0, semg0)

    def pair_body(i, carry):
        cha = 2 * i
        prefetch(cha + 1, idx1, cat1, semg1)
        compute(cha, idx0, cat0, semg0)

        @pl.when(i < NP - 1)
        def _prefetch_next_pair():
            prefetch(cha + 2, idx0, cat0, semg0)

        compute(cha + 1, idx1, cat1, semg1)
        return carry

    lax.fori_loop(0, NP, pair_body, 0)

    bl = (base + (NCH - 1) * CB) * ROW
    pltpu.make_async_copy(out_v, outh.at[pl.ds(bl, CB * ROW)], semo).wait()


def kernel(x_num, x_cat, weight, bias, emb_table, category_offsets):
    idx = x_cat.astype(jnp.int32) + category_offsets[None, :].astype(jnp.int32)
    xn = jnp.pad(x_num, ((0, 0), (0, 16 - D_NUM)))
    out = _tokenizer_sc(emb_table, idx.reshape(-1), xn, weight, bias)
    return out.reshape(B, N_TOK, D_TOKEN)
